# baseline (speedup 1.0000x reference)
import functools

import jax
import jax.numpy as jnp
from jax import lax
from jax.experimental import pallas as pl
from jax.experimental.pallas import tpu as pltpu
from jax.experimental.pallas import tpu_sc as plsc

_D = 64
_NBUF = 4


@functools.cache
def _make_gather(batch, seq):
    info = plsc.get_sparse_core_info()
    nc, ns = info.num_cores, info.num_subcores
    nw = nc * ns
    rows_per_w = batch // nw
    n_sub = 5
    sub = seq // n_sub
    mesh = plsc.VectorSubcoreMesh(core_axis_name="c", subcore_axis_name="s")

    @functools.partial(
        pl.kernel,
        out_type=jax.ShapeDtypeStruct((batch, seq, _D), jnp.float32),
        mesh=mesh,
        scratch_types=[
            pltpu.VMEM((rows_per_w, seq), jnp.int32),
            pltpu.VMEM((_NBUF, seq, _D), jnp.float32),
            pltpu.SemaphoreType.DMA((_NBUF,)),
            pltpu.SemaphoreType.DMA((_NBUF,)),
        ],
        compiler_params=pltpu.CompilerParams(use_tc_tiling_on_sc=False),
    )
    def gather_k(idx_hbm, table_hbm, out_hbm, idx_v, bufs, gsem, wsem):
        wid = lax.axis_index("s") * nc + lax.axis_index("c")
        row0 = wid * rows_per_w
        pltpu.sync_copy(idx_hbm.at[pl.ds(row0, rows_per_w)], idx_v)

        def fire_gathers(r, b):
            for h in range(n_sub):
                pltpu.async_copy(
                    table_hbm.at[idx_v.at[r, pl.ds(h * sub, sub)]],
                    bufs.at[b, pl.ds(h * sub, sub)], gsem.at[b])

        def wait_gathers(r, b):
            for h in range(n_sub):
                pltpu.make_async_copy(
                    table_hbm.at[idx_v.at[r, pl.ds(h * sub, sub)]],
                    bufs.at[b, pl.ds(h * sub, sub)], gsem.at[b]).wait()

        def fire_wb(r, b):
            pltpu.async_copy(bufs.at[b], out_hbm.at[row0 + r], wsem.at[b])

        def wait_wb(r, b):
            pltpu.make_async_copy(
                bufs.at[b], out_hbm.at[row0 + r], wsem.at[b]).wait()

        for b in range(_NBUF):
            fire_gathers(b, b)

        @pl.loop(0, rows_per_w - _NBUF, step=_NBUF)
        def _round(rr):
            for b in range(_NBUF):
                wait_gathers(rr + b, b)
                fire_wb(rr + b, b)
            for b in range(_NBUF):
                wait_wb(rr + b, b)
                fire_gathers(rr + _NBUF + b, b)

        last = rows_per_w - _NBUF
        for b in range(_NBUF):
            wait_gathers(last + b, b)
            fire_wb(last + b, b)
        for b in range(_NBUF):
            wait_wb(last + b, b)

    return gather_k


def kernel(word_inputs, table):
    batch, seq = word_inputs.shape
    return _make_gather(batch, seq)(word_inputs.astype(jnp.int32), table)


# speedup vs baseline: 1.2180x; 1.2180x over previous
import functools

import jax
import jax.numpy as jnp
from jax import lax
from jax.experimental import pallas as pl
from jax.experimental.pallas import tpu as pltpu
from jax.experimental.pallas import tpu_sc as plsc

_D = 64
_NBUF = 4


@functools.cache
def _make_gather(batch, seq):
    info = plsc.get_sparse_core_info()
    nc, ns = info.num_cores, info.num_subcores
    nw = nc * ns
    rows_per_w = batch // nw
    n_sub = 5
    sub = seq // n_sub
    mesh = plsc.VectorSubcoreMesh(core_axis_name="c", subcore_axis_name="s")

    @functools.partial(
        pl.kernel,
        out_type=jax.ShapeDtypeStruct((batch, seq, 128), jnp.float32),
        mesh=mesh,
        scratch_types=[
            pltpu.VMEM((rows_per_w, seq), jnp.int32),
            pltpu.VMEM((_NBUF, seq, 128), jnp.float32),
            pltpu.SemaphoreType.DMA((_NBUF,)),
            pltpu.SemaphoreType.DMA((_NBUF,)),
        ],
        compiler_params=pltpu.CompilerParams(use_tc_tiling_on_sc=False),
    )
    def gather_k(idx_hbm, tabp_hbm, out_hbm, idx_v, bufs, gsem, wsem):
        wid = lax.axis_index("s") * nc + lax.axis_index("c")
        row0 = wid * rows_per_w
        pltpu.sync_copy(idx_hbm.at[pl.ds(row0, rows_per_w)], idx_v)

        def fire_gathers(r, b):
            for h in range(n_sub):
                pltpu.async_copy(
                    tabp_hbm.at[idx_v.at[r, pl.ds(h * sub, sub)]],
                    bufs.at[b, pl.ds(h * sub, sub)], gsem.at[b])

        def wait_gathers(r, b):
            for h in range(n_sub):
                pltpu.make_async_copy(
                    tabp_hbm.at[idx_v.at[r, pl.ds(h * sub, sub)]],
                    bufs.at[b, pl.ds(h * sub, sub)], gsem.at[b]).wait()

        def fire_wb(r, b):
            pltpu.async_copy(bufs.at[b], out_hbm.at[row0 + r], wsem.at[b])

        def wait_wb(r, b):
            pltpu.make_async_copy(
                bufs.at[b], out_hbm.at[row0 + r], wsem.at[b]).wait()

        for b in range(_NBUF):
            fire_gathers(b, b)

        @pl.loop(0, rows_per_w - _NBUF, step=_NBUF)
        def _round(rr):
            for b in range(_NBUF):
                wait_gathers(rr + b, b)
                fire_wb(rr + b, b)
            for b in range(_NBUF):
                wait_wb(rr + b, b)
                fire_gathers(rr + _NBUF + b, b)

        last = rows_per_w - _NBUF
        for b in range(_NBUF):
            wait_gathers(last + b, b)
            fire_wb(last + b, b)
        for b in range(_NBUF):
            wait_wb(last + b, b)

    return gather_k


def kernel(word_inputs, table):
    batch, seq = word_inputs.shape
    tabp = jnp.pad(table, ((0, 0), (0, 128 - _D)))
    out = _make_gather(batch, seq)(word_inputs.astype(jnp.int32), tabp)
    return out[..., :_D]


# speedup vs baseline: 1.3313x; 1.0930x over previous
import functools

import jax
import jax.numpy as jnp
from jax import lax
from jax.experimental import pallas as pl
from jax.experimental.pallas import tpu as pltpu
from jax.experimental.pallas import tpu_sc as plsc

_D = 64
_NBUF = 4


@functools.cache
def _make_gather(batch, seq):
    info = plsc.get_sparse_core_info()
    nc, ns = info.num_cores, info.num_subcores
    nw = nc * ns
    rows_per_w = batch // nw
    n_sub = 5
    sub = seq // n_sub
    mesh = plsc.VectorSubcoreMesh(core_axis_name="c", subcore_axis_name="s")

    @functools.partial(
        pl.kernel,
        out_type=jax.ShapeDtypeStruct((batch, seq, 128), jnp.float32),
        mesh=mesh,
        scratch_types=[
            pltpu.VMEM((rows_per_w, seq), jnp.int32),
            pltpu.VMEM((_NBUF, seq, _D), jnp.float32),
            pltpu.SemaphoreType.DMA((_NBUF,)),
            pltpu.SemaphoreType.DMA((_NBUF,)),
        ],
        compiler_params=pltpu.CompilerParams(use_tc_tiling_on_sc=False),
    )
    def gather_k(idx_hbm, table_hbm, out_hbm, idx_v, bufs, gsem, wsem):
        wid = lax.axis_index("s") * nc + lax.axis_index("c")
        row0 = wid * rows_per_w
        pltpu.sync_copy(idx_hbm.at[pl.ds(row0, rows_per_w)], idx_v)

        def fire_gathers(r, b):
            for h in range(n_sub):
                pltpu.async_copy(
                    table_hbm.at[idx_v.at[r, pl.ds(h * sub, sub)]],
                    bufs.at[b, pl.ds(h * sub, sub)], gsem.at[b])

        def wait_gathers(r, b):
            for h in range(n_sub):
                pltpu.make_async_copy(
                    table_hbm.at[idx_v.at[r, pl.ds(h * sub, sub)]],
                    bufs.at[b, pl.ds(h * sub, sub)], gsem.at[b]).wait()

        def fire_wb(r, b):
            pltpu.async_copy(
                bufs.at[b], out_hbm.at[row0 + r, :, pl.ds(0, _D)],
                wsem.at[b])

        def wait_wb(r, b):
            pltpu.make_async_copy(
                bufs.at[b], out_hbm.at[row0 + r, :, pl.ds(0, _D)],
                wsem.at[b]).wait()

        for b in range(_NBUF):
            fire_gathers(b, b)

        @pl.loop(0, rows_per_w - _NBUF, step=_NBUF)
        def _round(rr):
            for b in range(_NBUF):
                wait_gathers(rr + b, b)
                fire_wb(rr + b, b)
            for b in range(_NBUF):
                wait_wb(rr + b, b)
                fire_gathers(rr + _NBUF + b, b)

        last = rows_per_w - _NBUF
        for b in range(_NBUF):
            wait_gathers(last + b, b)
            fire_wb(last + b, b)
        for b in range(_NBUF):
            wait_wb(last + b, b)

    return gather_k


def kernel(word_inputs, table):
    batch, seq = word_inputs.shape
    out = _make_gather(batch, seq)(word_inputs.astype(jnp.int32), table)
    return out[..., :_D]
